# baseline (speedup 1.0000x reference)
import functools

import jax
import jax.numpy as jnp
from jax.experimental import pallas as pl

N = 10000
E = 32000
D = 2560
H = 8
C = 320
G = 16
HC = H * C

BN = 256
BC = 512


def _matmul_body(x_ref, w_ref, b_ref, o_ref):
    acc = jnp.dot(x_ref[...], w_ref[...], preferred_element_type=jnp.float32)
    o_ref[...] = acc + b_ref[0:1, :]


def _proj(x, W, b2d, n_out):
    n_blocks = pl.cdiv(N, BN)
    c_blocks = n_out // BC
    return pl.pallas_call(
        _matmul_body,
        grid=(c_blocks, n_blocks),
        in_specs=[
            pl.BlockSpec((BN, D), lambda ch, n: (n, 0)),
            pl.BlockSpec((D, BC), lambda ch, n: (0, ch)),
            pl.BlockSpec((8, BC), lambda ch, n: (0, ch)),
        ],
        out_specs=pl.BlockSpec((BN, BC), lambda ch, n: (n, ch)),
        out_shape=jax.ShapeDtypeStruct((N, n_out), jnp.float32),
    )(x, W, b2d)


def _gatv2_layer(x, src, dst, ea, Wl, bl, Wr, br, We, att, bo):
    Wcat = jnp.concatenate([Wl, Wr], axis=1)
    bcat = jnp.broadcast_to(jnp.concatenate([bl, br])[None, :], (8, 2 * HC))
    y = _proj(x, Wcat, bcat, 2 * HC)
    xl = y[:, :HC].reshape(N, H, C)
    xr = y[:, HC:].reshape(N, H, C)
    ee = (ea @ We).reshape(-1, H, C)
    m = jax.nn.leaky_relu(xl[src] + xr[dst] + ee, negative_slope=0.2)
    logits = jnp.sum(m * att[None, :, :], axis=-1)
    lmax = jax.ops.segment_max(logits, dst, num_segments=N)
    lmax = jnp.where(jnp.isfinite(lmax), lmax, 0.0)
    ex = jnp.exp(logits - lmax[dst])
    denom = jax.ops.segment_sum(ex, dst, num_segments=N)
    alpha = ex / (denom[dst] + 1e-16)
    out = jax.ops.segment_sum(xl[src] * alpha[:, :, None], dst, num_segments=N)
    return out.reshape(-1, HC) + bo


def kernel(x, edge_index, edge_attr, batch, Wl1, bl1, Wr1, br1, We1, att1, bo1,
           Wl2, bl2, Wr2, br2, We2, att2, bo2):
    src = edge_index[0]
    dst = edge_index[1]
    h = _gatv2_layer(x, src, dst, edge_attr, Wl1, bl1, Wr1, br1, We1, att1, bo1)
    h = _gatv2_layer(h, src, dst, edge_attr, Wl2, bl2, Wr2, br2, We2, att2, bo2)
    counts = jax.ops.segment_sum(jnp.ones((N,), jnp.float32), batch, num_segments=G)
    sums = jax.ops.segment_sum(h, batch, num_segments=G)
    return sums / jnp.clip(counts, 1.0)[:, None]
